# baseline (speedup 1.0000x reference)
import jax
import jax.numpy as jnp
from jax import lax
from jax.experimental import pallas as pl
from jax.experimental.pallas import tpu as pltpu
from jax.experimental.pallas import tpu_sc as plsc

N_CLASSES = 1000
IMAGES_PER_CLASS = 10
IM_DIM = 32
IM_CHAN = 3
BATCH = 1024
ROW = IM_DIM * IM_DIM * IM_CHAN

_INFO = plsc.get_sparse_core_info()
_NC = _INFO.num_cores
_NS = _INFO.num_subcores
_NW = _NC * _NS
_BPW = BATCH // _NW
_LANES = _INFO.num_lanes


def _body(noise_hbm, labels_hbm, table_hbm, out_hbm, lab_v, noi_v, idx_v,
          rows_v, sem):
  wid = lax.axis_index("s") * _NC + lax.axis_index("c")
  base = wid * _BPW
  pltpu.sync_copy(labels_hbm.at[pl.ds(base, _BPW)], lab_v)
  pltpu.sync_copy(noise_hbm.at[pl.ds(base, _BPW)], noi_v)
  for j in range(_BPW // _LANES):
    sl = pl.ds(j * _LANES, _LANES)
    idx_v[sl] = lab_v[sl] * IMAGES_PER_CLASS + noi_v[sl]
  pltpu.async_copy(table_hbm.at[idx_v], rows_v, sem).wait()
  pltpu.sync_copy(rows_v, out_hbm.at[pl.ds(base, _BPW)])


@jax.jit
def _gather(noise, labels, table):
  mesh = plsc.VectorSubcoreMesh(core_axis_name="c", subcore_axis_name="s")
  return pl.kernel(
      _body,
      out_type=jax.ShapeDtypeStruct((BATCH, ROW), jnp.float32),
      mesh=mesh,
      scratch_types=[
          pltpu.VMEM((_BPW,), jnp.int32),
          pltpu.VMEM((_BPW,), jnp.int32),
          pltpu.VMEM((_BPW,), jnp.int32),
          pltpu.VMEM((_BPW, ROW), jnp.float32),
          pltpu.SemaphoreType.DMA,
      ],
  )(noise, labels, table)


def kernel(noise, labels, batches, is_training, data):
  table = data.reshape(N_CLASSES * IMAGES_PER_CLASS, ROW)
  out = _gather(noise, labels, table)
  return out.reshape(BATCH, IM_DIM, IM_DIM, IM_CHAN)
